# baseline (speedup 1.0000x reference)
import functools

import jax
import jax.numpy as jnp
from jax import lax
from jax.experimental import pallas as pl

_MIN_DISTANCE = 3
_THRESHOLD_REL = 0.01
_MAX_NUM_PEAKS = 100
_NEG = float("-inf")
_IBIG = 0x7FFFFFFF


def _shift(a, d, axis, fill):
    n = a.shape[axis]
    pad_shape = list(a.shape)
    pad_shape[axis] = abs(d)
    pad = jnp.full(pad_shape, fill, a.dtype)
    if d > 0:
        body = jax.lax.slice_in_dim(a, d, n, axis=axis)
        return jax.lax.concatenate([body, pad], axis)
    else:
        body = jax.lax.slice_in_dim(a, 0, n + d, axis=axis)
        return jax.lax.concatenate([pad, body], axis)


def _pool7(v, axis):
    w3 = jnp.maximum(jnp.maximum(_shift(v, 1, axis, _NEG), _shift(v, -1, axis, _NEG)), v)
    return jnp.maximum(jnp.maximum(_shift(w3, 2, axis, _NEG), _shift(w3, -2, axis, _NEG)), w3)


def _argmax_step(val, idx, d, axis):
    sv = _shift(val, d, axis, _NEG)
    si = _shift(idx, d, axis, jnp.int32(0))
    take = sv > val
    return jnp.maximum(val, sv), jnp.where(take, si, idx)


def _dense_kernel(x_ref, bval_ref, bidx_ref):
    v = x_ref[0]
    H, W = v.shape
    m = jnp.max(v)
    e = jnp.exp(v - m)
    se = jnp.sum(e)
    q = e / se
    maxq = jnp.max(q)
    thr_abs = 1.0 / (H * W) * 2.0
    thr_rel = _THRESHOLD_REL * maxq

    pooled = _pool7(_pool7(q, 0), 1)
    peak = (q == pooled) & (q > thr_abs) & (q > thr_rel)
    masked = jnp.where(peak, q, 0.0)

    lin = (jax.lax.broadcasted_iota(jnp.int32, (H, W), 0) * W
           + jax.lax.broadcasted_iota(jnp.int32, (H, W), 1))

    val, idx = _argmax_step(masked, lin, 1, 1)
    val, idx = _argmax_step(val, idx, 2, 1)
    val, idx = _argmax_step(val, idx, 1, 0)
    val, idx = _argmax_step(val, idx, 2, 0)

    hb, wb = H // 4, W // 4
    selr = (jax.lax.broadcasted_iota(jnp.int32, (hb, H), 1)
            == 4 * jax.lax.broadcasted_iota(jnp.int32, (hb, H), 0)
            ).astype(jnp.float32)
    selc = (jax.lax.broadcasted_iota(jnp.int32, (W, wb), 0)
            == 4 * jax.lax.broadcasted_iota(jnp.int32, (W, wb), 1)
            ).astype(jnp.float32)

    def _sel(a):
        t = jax.lax.dot_general(a, selc, (((1,), (0,)), ((), ())),
                                precision=jax.lax.Precision.HIGHEST,
                                preferred_element_type=jnp.float32)
        return jax.lax.dot_general(selr, t, (((1,), (0,)), ((), ())),
                                   precision=jax.lax.Precision.HIGHEST,
                                   preferred_element_type=jnp.float32)

    bval_ref[0] = _sel(val)
    bidx_ref[0] = _sel(idx.astype(jnp.float32)).astype(jnp.int32)


def _dense_stage(x):
    C, H, W = x.shape
    hb, wb = H // 4, W // 4
    return pl.pallas_call(
        _dense_kernel,
        grid=(C,),
        in_specs=[pl.BlockSpec((1, H, W), lambda i: (i, 0, 0))],
        out_specs=[
            pl.BlockSpec((1, hb, wb), lambda i: (i, 0, 0)),
            pl.BlockSpec((1, hb, wb), lambda i: (i, 0, 0)),
        ],
        out_shape=[
            jax.ShapeDtypeStruct((C, hb, wb), jnp.float32),
            jax.ShapeDtypeStruct((C, hb, wb), jnp.int32),
        ],
    )(x)


def _select_kernel(bval_ref, bidx_ref, vout_ref, iout_ref):
    a = bval_ref[0]
    ai = bidx_ref[0]

    rows = jax.lax.broadcasted_iota(jnp.int32, (8, 128), 0)
    cols = jax.lax.broadcasted_iota(jnp.int32, (8, 128), 1)

    def body(r, carry):
        a, winv, wini = carry
        mx = jnp.max(a)
        mn = jnp.min(jnp.where(a == mx, ai, _IBIG))
        hit = (a == mx) & (ai == mn)
        a = jnp.where(hit, 0.0, a)
        ok = mx > 0.0
        slot = (rows == r // 128) & (cols == r % 128)
        winv = jnp.where(slot & ok, mx, winv)
        wini = jnp.where(slot & ok, mn, wini)
        return a, winv, wini

    _, winv, wini = lax.fori_loop(
        0, _MAX_NUM_PEAKS, body,
        (a, jnp.zeros((8, 128), jnp.float32), jnp.zeros((8, 128), jnp.int32)))
    vout_ref[0] = winv
    iout_ref[0] = wini


def _select_stage(bval, bidx):
    C = bval.shape[0]
    return pl.pallas_call(
        _select_kernel,
        grid=(C,),
        in_specs=[pl.BlockSpec((1, 128, 128), lambda i: (i, 0, 0)),
                  pl.BlockSpec((1, 128, 128), lambda i: (i, 0, 0))],
        out_specs=[pl.BlockSpec((1, 8, 128), lambda i: (i, 0, 0)),
                   pl.BlockSpec((1, 8, 128), lambda i: (i, 0, 0))],
        out_shape=[jax.ShapeDtypeStruct((C, 8, 128), jnp.float32),
                   jax.ShapeDtypeStruct((C, 8, 128), jnp.int32)],
    )(bval, bidx)


def kernel(belive_map):
    B, S, H, W = belive_map.shape
    x = belive_map.reshape(B * S, H, W)
    bval, bidx = _dense_stage(x)
    outv, outi = _select_stage(bval, bidx)

    vals = outv.reshape(B * S, -1)[:, :_MAX_NUM_PEAKS].reshape(B, S, _MAX_NUM_PEAKS)
    idx = outi.reshape(B * S, -1)[:, :_MAX_NUM_PEAKS].reshape(B, S, _MAX_NUM_PEAKS)
    valid = vals > 0.0
    scores = jnp.where(valid, vals, 0.0)
    rows = idx // W
    cols = idx % W
    seg = jnp.broadcast_to(jnp.arange(S, dtype=idx.dtype)[None, :, None],
                           (B, S, _MAX_NUM_PEAKS))
    skeletons = jnp.stack([seg, cols, rows], axis=-1)
    return skeletons, scores.astype(jnp.float32), valid


# speedup vs baseline: 1.0086x; 1.0086x over previous
import functools

import jax
import jax.numpy as jnp
from jax import lax
from jax.experimental import pallas as pl
from jax.experimental.pallas import tpu as pltpu
from jax.experimental.pallas import tpu_sc as plsc

_MIN_DISTANCE = 3
_THRESHOLD_REL = 0.01
_MAX_NUM_PEAKS = 100
_NEG = float("-inf")
_IBIG = 0x7FFFFFFF


def _shift(a, d, axis, fill):
    n = a.shape[axis]
    pad_shape = list(a.shape)
    pad_shape[axis] = abs(d)
    pad = jnp.full(pad_shape, fill, a.dtype)
    if d > 0:
        body = jax.lax.slice_in_dim(a, d, n, axis=axis)
        return jax.lax.concatenate([body, pad], axis)
    else:
        body = jax.lax.slice_in_dim(a, 0, n + d, axis=axis)
        return jax.lax.concatenate([pad, body], axis)


def _pool7(v, axis):
    w3 = jnp.maximum(jnp.maximum(_shift(v, 1, axis, _NEG), _shift(v, -1, axis, _NEG)), v)
    return jnp.maximum(jnp.maximum(_shift(w3, 2, axis, _NEG), _shift(w3, -2, axis, _NEG)), w3)


def _argmax_step(val, idx, d, axis):
    sv = _shift(val, d, axis, _NEG)
    si = _shift(idx, d, axis, jnp.int32(0))
    take = sv > val
    return jnp.maximum(val, sv), jnp.where(take, si, idx)


def _dense_kernel(x_ref, bval_ref, bidx_ref):
    v = x_ref[0]
    H, W = v.shape
    m = jnp.max(v)
    e = jnp.exp(v - m)
    se = jnp.sum(e)
    q = e / se
    maxq = jnp.max(q)
    thr_abs = 1.0 / (H * W) * 2.0
    thr_rel = _THRESHOLD_REL * maxq

    pooled = _pool7(_pool7(q, 0), 1)
    peak = (q == pooled) & (q > thr_abs) & (q > thr_rel)
    masked = jnp.where(peak, q, 0.0)

    lin = (jax.lax.broadcasted_iota(jnp.int32, (H, W), 0) * W
           + jax.lax.broadcasted_iota(jnp.int32, (H, W), 1))

    val, idx = _argmax_step(masked, lin, 1, 1)
    val, idx = _argmax_step(val, idx, 2, 1)
    val, idx = _argmax_step(val, idx, 1, 0)
    val, idx = _argmax_step(val, idx, 2, 0)

    hb, wb = H // 4, W // 4
    selr = (jax.lax.broadcasted_iota(jnp.int32, (hb, H), 1)
            == 4 * jax.lax.broadcasted_iota(jnp.int32, (hb, H), 0)
            ).astype(jnp.float32)
    selc = (jax.lax.broadcasted_iota(jnp.int32, (W, wb), 0)
            == 4 * jax.lax.broadcasted_iota(jnp.int32, (W, wb), 1)
            ).astype(jnp.float32)

    def _sel(a):
        t = jax.lax.dot_general(a, selc, (((1,), (0,)), ((), ())),
                                precision=jax.lax.Precision.HIGHEST,
                                preferred_element_type=jnp.float32)
        return jax.lax.dot_general(selr, t, (((1,), (0,)), ((), ())),
                                   precision=jax.lax.Precision.HIGHEST,
                                   preferred_element_type=jnp.float32)

    bval_ref[0] = _sel(val)
    bidx_ref[0] = _sel(idx.astype(jnp.float32)).astype(jnp.int32)


def _dense_stage(x):
    C, H, W = x.shape
    hb, wb = H // 4, W // 4
    return pl.pallas_call(
        _dense_kernel,
        grid=(C,),
        in_specs=[pl.BlockSpec((1, H, W), lambda i: (i, 0, 0))],
        out_specs=[
            pl.BlockSpec((1, hb, wb), lambda i: (i, 0, 0)),
            pl.BlockSpec((1, hb, wb), lambda i: (i, 0, 0)),
        ],
        out_shape=[
            jax.ShapeDtypeStruct((C, hb, wb), jnp.float32),
            jax.ShapeDtypeStruct((C, hb, wb), jnp.int32),
        ],
    )(x)


_N_WORKERS = 32
_NBINS = 1024


def _lanes():
    return lax.iota(jnp.int32, 16)


def _splat(x, dtype):
    return jnp.zeros((16,), dtype) + x


def _sc_sel_kernel(bval_hbm, bidx_hbm, outv_hbm, outi_hbm,
                   val2_v, idx2_v, cval_v, cidx_v, sval_v, sidx_v,
                   hist_v, ov_v, oi_v):
    C = bval_hbm.shape[0]
    wid = lax.axis_index("s") * 2 + lax.axis_index("c")
    n_rounds = (C + _N_WORKERS - 1) // _N_WORKERS

    for i in range(n_rounds):
        c = wid + i * _N_WORKERS

        @pl.when(c < C)
        def _process():
            pltpu.sync_copy(bval_hbm.at[c], val2_v)
            pltpu.sync_copy(bidx_hbm.at[c], idx2_v)

            def scan_body(j, carry):
                cur, mnb, mxb = carry
                r, jj = j // 8, j % 8
                v = val2_v[r, pl.ds(jj * 16, 16)]
                ix = idx2_v[r, pl.ds(jj * 16, 16)]
                msk = v > 0.0
                cs = plsc.cumsum(msk.astype(jnp.int32))
                pos = cur + cs - 1
                plsc.store_scatter(cval_v, [pos], v, mask=msk)
                plsc.store_scatter(cidx_v, [pos], ix, mask=msk)
                bb = plsc.bitcast(v, jnp.int32)
                mnb = jnp.minimum(mnb, jnp.min(jnp.where(msk, bb, _IBIG)))
                mxb = jnp.maximum(mxb, jnp.max(jnp.where(msk, bb, 0)))
                return cur + jnp.max(cs), mnb, mxb

            n, mnb, mxb = lax.fori_loop(
                0, 1024, scan_body,
                (jnp.int32(0), jnp.int32(_IBIG), jnp.int32(0)))

            rng = mxb - mnb
            k = lax.while_loop(
                lambda kk: (rng >> kk) >= _NBINS, lambda kk: kk + 1,
                jnp.int32(0))

            def zero_hist(j, _):
                hist_v[pl.ds(j * 16, 16)] = jnp.zeros((16,), jnp.int32)
                return 0
            lax.fori_loop(0, _NBINS // 16, zero_hist, 0)

            nv = (n + 15) // 16

            def hist_body(j, _):
                v = cval_v[pl.ds(j * 16, 16)]
                msk = _lanes() < (n - j * 16)
                bb = plsc.bitcast(v, jnp.int32)
                bins = (bb - mnb) >> k
                plsc.addupdate_scatter(hist_v, [bins],
                                       jnp.ones((16,), jnp.int32), mask=msk)
                return 0
            lax.fori_loop(0, nv, hist_body, 0)

            def cut_body(j, carry):
                acc, bstar = carry
                jj = _NBINS // 16 - 1 - j
                h = hist_v[pl.ds(jj * 16, 16)]
                s = jnp.sum(h)
                rev = lax.rev(h, (0,))
                cs = acc + plsc.cumsum(rev)
                cross = cs >= _MAX_NUM_PEAKS
                f = jnp.max(plsc.all_reduce_ffs(cross))
                hit = (bstar < 0) & (acc + s >= _MAX_NUM_PEAKS)
                bnew = jj * 16 + 15 - f
                bstar = jnp.where(hit, bnew, bstar)
                acc = jnp.where(bstar < 0, acc + s, acc)
                return acc, bstar

            _, bstar = lax.fori_loop(0, _NBINS // 16, cut_body,
                                     (jnp.int32(0), jnp.int32(-1)))
            bstar = jnp.maximum(bstar, 0)
            cutbits = mnb + (bstar << k)

            def sel_body(j, cur):
                v = cval_v[pl.ds(j * 16, 16)]
                ix = cidx_v[pl.ds(j * 16, 16)]
                bb = plsc.bitcast(v, jnp.int32)
                msk = (_lanes() < (n - j * 16)) & (bb >= cutbits)
                cs = plsc.cumsum(msk.astype(jnp.int32))
                pos = cur + cs - 1
                plsc.store_scatter(sval_v, [pos], v, mask=msk)
                plsc.store_scatter(sidx_v, [pos], ix, mask=msk)
                return cur + jnp.max(cs)
            m = lax.fori_loop(0, nv, sel_body, jnp.int32(0))
            mv = (m + 15) // 16

            def round_body(r, _):
                def find_body(j, carry):
                    bv, bi, bp = carry
                    v = sval_v[pl.ds(j * 16, 16)]
                    ix = sidx_v[pl.ds(j * 16, 16)]
                    lm = _lanes() < (m - j * 16)
                    vv = jnp.where(lm, v, -1.0)
                    mx = jnp.max(vv)
                    tie = vv == mx
                    ii = jnp.where(tie, ix, _IBIG)
                    mn = jnp.min(ii)
                    f = jnp.max(plsc.all_reduce_ffs(ii == mn))
                    pos = j * 16 + f
                    better = (mx > bv) | ((mx == bv) & (mn < bi))
                    return (jnp.where(better, mx, bv),
                            jnp.where(better, mn, bi),
                            jnp.where(better, pos, bp))

                bv, bi, bp = lax.fori_loop(
                    0, mv, find_body,
                    (jnp.float32(-1.0), jnp.int32(_IBIG), jnp.int32(0)))
                ok = bv > 0.0
                lane0 = _lanes() == 0
                plsc.store_scatter(ov_v, [_splat(r, jnp.int32)],
                                   _splat(jnp.where(ok, bv, 0.0), jnp.float32),
                                   mask=lane0)
                plsc.store_scatter(oi_v, [_splat(r, jnp.int32)],
                                   _splat(jnp.where(ok, bi, 0), jnp.int32),
                                   mask=lane0)
                plsc.store_scatter(sval_v, [_splat(bp, jnp.int32)],
                                   jnp.zeros((16,), jnp.float32),
                                   mask=lane0 & ok)
                return 0

            lax.fori_loop(0, _MAX_NUM_PEAKS, round_body, 0)
            pltpu.sync_copy(ov_v, outv_hbm.at[c])
            pltpu.sync_copy(oi_v, outi_hbm.at[c])


def _select_stage_sc(bval, bidx):
    C = bval.shape[0]
    mesh = plsc.VectorSubcoreMesh(core_axis_name="c", subcore_axis_name="s")
    f = functools.partial(
        pl.kernel, mesh=mesh,
        out_type=[jax.ShapeDtypeStruct((C, 128), jnp.float32),
                  jax.ShapeDtypeStruct((C, 128), jnp.int32)],
        scratch_types=[
            pltpu.VMEM((128, 128), jnp.float32),
            pltpu.VMEM((128, 128), jnp.int32),
            pltpu.VMEM((16384 + 16,), jnp.float32),
            pltpu.VMEM((16384 + 16,), jnp.int32),
            pltpu.VMEM((16384 + 16,), jnp.float32),
            pltpu.VMEM((16384 + 16,), jnp.int32),
            pltpu.VMEM((_NBINS,), jnp.int32),
            pltpu.VMEM((128,), jnp.float32),
            pltpu.VMEM((128,), jnp.int32),
        ],
    )(_sc_sel_kernel)
    return f(bval, bidx)


def _select_kernel(bval_ref, bidx_ref, vout_ref, iout_ref):
    a = bval_ref[0]
    ai = bidx_ref[0]

    rows = jax.lax.broadcasted_iota(jnp.int32, (8, 128), 0)
    cols = jax.lax.broadcasted_iota(jnp.int32, (8, 128), 1)

    def reduce_tree(a):
        parts = [(jax.lax.slice_in_dim(a, 8 * k, 8 * k + 8, axis=0),
                  jax.lax.slice_in_dim(ai, 8 * k, 8 * k + 8, axis=0))
                 for k in range(16)]
        while len(parts) > 1:
            nxt = []
            for (v1, i1), (v2, i2) in zip(parts[::2], parts[1::2]):
                take = (v2 > v1) | ((v2 == v1) & (i2 < i1))
                nxt.append((jnp.where(take, v2, v1), jnp.where(take, i2, i1)))
            parts = nxt
        return parts[0]

    def body(r, carry):
        a, rv, rix, winv, wini = carry
        mx = jnp.max(rv)
        mn = jnp.min(jnp.where(rv == mx, rix, _IBIG))
        ok = mx > 0.0
        slot = (rows == r // 128) & (cols == r % 128)
        winv = jnp.where(slot & ok, mx, winv)
        wini = jnp.where(slot & ok, mn, wini)
        a = jnp.where(ai == mn, 0.0, a)
        rv, rix = reduce_tree(a)
        return a, rv, rix, winv, wini

    rv0, rix0 = reduce_tree(a)
    _, _, _, winv, wini = lax.fori_loop(
        0, _MAX_NUM_PEAKS, body,
        (a, rv0, rix0,
         jnp.zeros((8, 128), jnp.float32), jnp.zeros((8, 128), jnp.int32)))
    vout_ref[0] = winv
    iout_ref[0] = wini


def _select_stage(bval, bidx):
    C = bval.shape[0]
    return pl.pallas_call(
        _select_kernel,
        grid=(C,),
        in_specs=[pl.BlockSpec((1, 128, 128), lambda i: (i, 0, 0)),
                  pl.BlockSpec((1, 128, 128), lambda i: (i, 0, 0))],
        out_specs=[pl.BlockSpec((1, 8, 128), lambda i: (i, 0, 0)),
                   pl.BlockSpec((1, 8, 128), lambda i: (i, 0, 0))],
        out_shape=[jax.ShapeDtypeStruct((C, 8, 128), jnp.float32),
                   jax.ShapeDtypeStruct((C, 8, 128), jnp.int32)],
    )(bval, bidx)


def kernel(belive_map):
    B, S, H, W = belive_map.shape
    x = belive_map.reshape(B * S, H, W)
    bval, bidx = _dense_stage(x)
    outv, outi = _select_stage(bval, bidx)

    vals = outv.reshape(B * S, -1)[:, :_MAX_NUM_PEAKS].reshape(B, S, _MAX_NUM_PEAKS)
    idx = outi.reshape(B * S, -1)[:, :_MAX_NUM_PEAKS].reshape(B, S, _MAX_NUM_PEAKS)
    valid = vals > 0.0
    scores = jnp.where(valid, vals, 0.0)
    rows = idx // W
    cols = idx % W
    seg = jnp.broadcast_to(jnp.arange(S, dtype=idx.dtype)[None, :, None],
                           (B, S, _MAX_NUM_PEAKS))
    skeletons = jnp.stack([seg, cols, rows], axis=-1)
    return skeletons, scores.astype(jnp.float32), valid
